# baseline (speedup 1.0000x reference)
import jax
import jax.numpy as jnp
from jax.experimental import pallas as pl
from jax.experimental.pallas import tpu as pltpu


def _stoch_attn_kernel(x_ref, wv_ref, out_ref):
    wv_sum = jnp.sum(wv_ref[...], axis=0)
    xb = x_ref[0]
    u = jnp.sum(xb * wv_sum[None, :], axis=1)
    out_ref[0] = jnp.broadcast_to(u[None, :], out_ref.shape[1:])


def kernel(x, idx, Wq, Wk, Wv, attention_scores):
    B, N, F = x.shape
    out = pl.pallas_call(
        _stoch_attn_kernel,
        grid=(B,),
        in_specs=[
            pl.BlockSpec((1, N, F), lambda b: (b, 0, 0)),
            pl.BlockSpec((F, F), lambda b: (0, 0)),
        ],
        out_specs=pl.BlockSpec((1, N, N), lambda b: (b, 0, 0)),
        out_shape=jax.ShapeDtypeStruct((B, N, N), jnp.float32),
        compiler_params=pltpu.CompilerParams(
            dimension_semantics=("parallel",),
        ),
    )(x, Wv)
    return out


# speedup vs baseline: 1.0482x; 1.0482x over previous
import jax
import jax.numpy as jnp
from jax.experimental import pallas as pl
from jax.experimental.pallas import tpu as pltpu


_BB = 2


def _stoch_attn_kernel(x_ref, wv_ref, out_ref):
    wv_sum = jnp.sum(wv_ref[...], axis=0)
    for i in range(_BB):
        xb = x_ref[i]
        u = jnp.sum(xb * wv_sum[None, :], axis=1)
        out_ref[i] = jnp.broadcast_to(u[None, :], out_ref.shape[1:])


def kernel(x, idx, Wq, Wk, Wv, attention_scores):
    B, N, F = x.shape
    out = pl.pallas_call(
        _stoch_attn_kernel,
        grid=(B // _BB,),
        in_specs=[
            pl.BlockSpec((_BB, N, F), lambda b: (b, 0, 0)),
            pl.BlockSpec((F, F), lambda b: (0, 0)),
        ],
        out_specs=pl.BlockSpec((_BB, N, N), lambda b: (b, 0, 0)),
        out_shape=jax.ShapeDtypeStruct((B, N, N), jnp.float32),
        compiler_params=pltpu.CompilerParams(
            dimension_semantics=("parallel",),
        ),
    )(x, Wv)
    return out
